# baseline (speedup 1.0000x reference)
import functools
import math

import jax
import jax.numpy as jnp
from jax import lax
from jax.experimental import pallas as pl
from jax.experimental.pallas import tpu as pltpu
from jax.experimental.pallas import tpu_sc as plsc

_ROW_BLOCK = 512
_SQRT2 = math.sqrt(2.0)
_N8 = 2000


def _gelu_exact(y):
    return 0.5 * y * (1.0 + lax.erf(y / _SQRT2))


def _ln_rows(y, g, b, eps=1e-5):
    m = jnp.mean(y, axis=-1, keepdims=True)
    v = jnp.mean((y - m) ** 2, axis=-1, keepdims=True)
    return (y - m) / jnp.sqrt(v + eps) * g + b


def _l2_rows(y, eps=1e-8):
    n = jnp.sqrt(jnp.sum(y * y, axis=-1, keepdims=True))
    return y / jnp.maximum(n, eps)


def _rowwise_body(flags, *refs):
    it = iter(refs)
    x_ref = next(it)
    y = x_ref[...]
    if flags["nan"]:
        y = jnp.nan_to_num(y)
    if flags["skip"]:
        y = y + next(it)[...]
    if flags["rs_in"]:
        y = y * next(it)[...]
    if flags["matmul"]:
        w = next(it)[...]
        y = jnp.dot(y, w, preferred_element_type=jnp.float32)
    if flags["bias"]:
        y = y + next(it)[...]
    if flags["ln"]:
        g = next(it)[...]
        beta = next(it)[...]
        y = _ln_rows(y, g, beta)
    if flags["gelu"]:
        y = _gelu_exact(y)
    if flags["rs_out"]:
        y = y * next(it)[...]
    if flags["l2"]:
        y = _l2_rows(y)
    out_ref = next(it)
    out_ref[...] = y


def _rowwise(x, *, skip=None, rs_in=None, W=None, b=None, g=None, beta=None,
             rs_out=None, nan=False, gelu=False, l2=False):
    n, k = x.shape
    m = W.shape[1] if W is not None else k
    flags = dict(nan=nan, skip=skip is not None, rs_in=rs_in is not None,
                 matmul=W is not None, bias=b is not None,
                 ln=g is not None, gelu=gelu, rs_out=rs_out is not None,
                 l2=l2)
    args = [x]
    specs = [pl.BlockSpec((_ROW_BLOCK, k), lambda i: (i, 0))]
    if skip is not None:
        args.append(skip)
        specs.append(pl.BlockSpec((_ROW_BLOCK, k), lambda i: (i, 0)))
    if rs_in is not None:
        args.append(rs_in.reshape(n, 1))
        specs.append(pl.BlockSpec((_ROW_BLOCK, 1), lambda i: (i, 0)))
    if W is not None:
        args.append(W)
        specs.append(pl.BlockSpec((k, m), lambda i: (0, 0)))
    if b is not None:
        args.append(b.reshape(1, m))
        specs.append(pl.BlockSpec((1, m), lambda i: (0, 0)))
    if g is not None:
        args.append(g.reshape(1, m))
        specs.append(pl.BlockSpec((1, m), lambda i: (0, 0)))
        args.append(beta.reshape(1, m))
        specs.append(pl.BlockSpec((1, m), lambda i: (0, 0)))
    if rs_out is not None:
        args.append(rs_out.reshape(n, 1))
        specs.append(pl.BlockSpec((_ROW_BLOCK, 1), lambda i: (i, 0)))
    return pl.pallas_call(
        functools.partial(_rowwise_body, flags),
        grid=(pl.cdiv(n, _ROW_BLOCK),),
        in_specs=specs,
        out_specs=pl.BlockSpec((_ROW_BLOCK, m), lambda i: (i, 0)),
        out_shape=jax.ShapeDtypeStruct((n, m), jnp.float32),
    )(*args)


def _combine_body(w_ref, a_ref, b_ref, o_ref):
    y = a_ref[...] * w_ref[0] + b_ref[...] * w_ref[1]
    o_ref[...] = _l2_rows(y)


def _combine2_l2(a, b, w):
    n, k = a.shape
    return pl.pallas_call(
        _combine_body,
        grid=(pl.cdiv(n, _ROW_BLOCK),),
        in_specs=[pl.BlockSpec(memory_space=pltpu.SMEM),
                  pl.BlockSpec((_ROW_BLOCK, k), lambda i: (i, 0)),
                  pl.BlockSpec((_ROW_BLOCK, k), lambda i: (i, 0))],
        out_specs=pl.BlockSpec((_ROW_BLOCK, k), lambda i: (i, 0)),
        out_shape=jax.ShapeDtypeStruct((n, k), jnp.float32),
    )(w, a, b)


_NS = 16
_NC = 2
_JB = 128


def _acc_rows(n_out):
    return -(-(n_out + 1) // 128) * 128


def _pick_ke(n_out):
    budget = 2097000 - _acc_rows(n_out) * 32
    ke = budget // (_NS * 35) // _JB * _JB
    return max(_JB, min(2048, ke))


def _make_edge_kernel(n_in, epad, acc_rows, ke):
    stripe = acc_rows // _NS
    ept = epad // _NS
    nchunks = ept // ke
    nj = ke // _JB
    mesh = plsc.VectorSubcoreMesh(core_axis_name="c", subcore_axis_name="s")

    def body(xs, src2, dst2, ewf, zrow,
             out, idx_v, dst_v, ew_v, rows_v, acc, sem):
        c = lax.axis_index("c")
        s = lax.axis_index("s")
        off = c
        pltpu.sync_copy(zrow, acc.at[pl.ds(s * stripe, stripe)])
        plsc.subcore_barrier()

        def chunk(i, carry):
            b128 = s * (ept // _JB) + i * nj
            bflat = s * ept + i * ke
            stage = [pltpu.async_copy(src2.at[pl.ds(b128, nj)], idx_v, sem),
                     pltpu.async_copy(dst2.at[pl.ds(b128, nj)], dst_v, sem),
                     pltpu.async_copy(ewf.at[pl.ds(bflat, ke)], ew_v, sem)]
            for cp in stage:
                cp.wait()

            @plsc.parallel_loop(0, nj * (_JB // 16), unroll=4)
            def _(v):
                j = v // (_JB // 16)
                sl = pl.ds((v % (_JB // 16)) * 16, 16)
                idx_v[j, sl] = idx_v[j, sl] + off

            cps = [pltpu.async_copy(xs.at[idx_v.at[j]],
                                    rows_v.at[pl.ds(j * _JB, _JB)], sem)
                   for j in range(nj)]
            for cp in cps:
                cp.wait()

            @plsc.parallel_loop(0, ke, unroll=32)
            def _(e):
                sp = plsc.load_gather(ew_v, [jnp.full((16,), e, jnp.int32)])
                rows_v[e, 0:16] = rows_v[e, 0:16] * sp
                rows_v[e, 16:32] = rows_v[e, 16:32] * sp

            sps = [pltpu.async_copy(rows_v.at[pl.ds(j * _JB, _JB)],
                                    acc.at[dst_v.at[j]], sem, add=True)
                   for j in range(nj)]
            for sp in sps:
                sp.wait()
            return carry
        lax.fori_loop(0, nchunks, chunk, 0)
        plsc.subcore_barrier()
        pltpu.sync_copy(acc.at[pl.ds(s * stripe, stripe)],
                        out.at[c].at[pl.ds(s * stripe, stripe)])

    return pl.kernel(
        body, mesh=mesh,
        compiler_params=pltpu.CompilerParams(needs_layout_passes=False,
                                             use_tc_tiling_on_sc=False),
        out_type=jax.ShapeDtypeStruct((_NC, acc_rows, 32), jnp.float32),
        scratch_types=[
            pltpu.VMEM((nj, _JB), jnp.int32),
            pltpu.VMEM((nj, _JB), jnp.int32),
            pltpu.VMEM((ke,), jnp.float32),
            pltpu.VMEM((ke, 32), jnp.float32),
            pltpu.VMEM_SHARED((acc_rows, 32), jnp.float32),
            pltpu.SemaphoreType.DMA,
        ])


def _prep_edges(src, dst, w, n_out):
    e = src.shape[0]
    quant = _NS * _pick_ke(n_out)
    epad = -(-e // quant) * quant
    pad = epad - e
    src2 = jnp.pad(2 * src, (0, pad)).reshape(-1, _JB)
    dst2 = jnp.pad(dst, (0, pad), constant_values=n_out).reshape(-1, _JB)
    wf = jnp.pad(w, (0, pad))
    return src2, dst2, wf


def _edge_pass(x, src2, dst2, wf, n_out):
    n_in = x.shape[0]
    acc_rows = _acc_rows(n_out)
    zrow = jnp.zeros((acc_rows // _NS, 32), jnp.float32)
    xs = x.reshape(2 * n_in, 32)
    out = _make_edge_kernel(n_in, wf.shape[0], acc_rows, _pick_ke(n_out))(
        xs, src2, dst2, wf, zrow)
    return jnp.concatenate([out[0, :n_out], out[1, :n_out]], axis=1)


def _degree(edges, n):
    src2, dst2, wf = edges
    ones = jnp.ones((n, 64), jnp.float32)
    return _edge_pass(ones, src2, dst2, wf, n)[:, 0]


def _dis(deg):
    return jnp.where(deg > 0, lax.rsqrt(jnp.maximum(deg, 1e-30)), 0.0)


def _block(x, edges, dis, p, n, skip=None):
    src2, dst2, wf = edges
    inp = x if skip is None else x + skip
    idt = _rowwise(inp, W=p['rW'], b=p['rb'], g=p['rg'], beta=p['rbeta'],
                   gelu=True)
    out = inp
    for i in range(4):
        y = _rowwise(out, W=p['cW'][i], rs_out=dis)
        z = _edge_pass(y, src2, dst2, wf, n)
        out = _rowwise(z, rs_in=dis, b=p['cb'][i], g=p['ng'][i],
                       beta=p['nb'][i], gelu=True)
    return _rowwise(out, skip=idt, l2=True)


def _spmap(x, src, dst, vals, n_out, p):
    src2, dst2, wf = _prep_edges(src, dst, vals, n_out)
    mapped = _edge_pass(x, src2, dst2, wf, n_out)
    return _rowwise(mapped, W=p['W'], b=p['b'], gelu=True, l2=True)


def kernel(feat_poi, feat_mobility, edge_index_10, edge_weight_10,
           edge_index_9, edge_weight_9, edge_index_8, edge_weight_8,
           map109_rows, map109_cols, map109_vals, map98_rows, map98_cols,
           map98_vals, params):
    p = params
    f = p['fusion']
    N10 = feat_poi.shape[0]
    N9 = map98_rows.shape[0]
    N8 = _N8

    row10, col10 = edge_index_10[0], edge_index_10[1]
    row9, col9 = edge_index_9[0], edge_index_9[1]
    row8, col8 = edge_index_8[0], edge_index_8[1]

    ed10 = _prep_edges(row10, col10, edge_weight_10, N10)
    ed9 = _prep_edges(row9, col9, edge_weight_9, N9)
    ed8 = _prep_edges(row8, col8, edge_weight_8, N8)

    dis10 = _dis(_degree(ed10, N10))
    dis9 = _dis(_degree(ed9, N9))
    dis8 = _dis(_degree(ed8, N8))

    proj_p = _rowwise(feat_poi, W=f['poi_W'], b=f['poi_b'], g=f['poi_g'],
                      beta=f['poi_beta'], nan=True, gelu=True, l2=True)
    proj_m = _rowwise(feat_mobility, W=f['mob_W'], b=f['mob_b'], g=f['mob_g'],
                      beta=f['mob_beta'], nan=True, gelu=True, l2=True)
    w = jax.nn.softmax(f['mw'])
    x = _combine2_l2(proj_p, proj_m, w)

    e1 = _block(x, ed10, dis10, p['enc1'], N10)
    e1m = _spmap(e1, map109_rows, map109_cols, map109_vals, N9, p['map'])
    e2 = _block(e1m, ed9, dis9, p['enc2'], N9)
    e2m = _spmap(e2, map98_rows, map98_cols, map98_vals, N8, p['map'])
    e3 = _block(e2m, ed8, dis8, p['enc3'], N8)
    d3 = _block(e3, ed8, dis8, p['dec3'], N8, skip=e3)
    d3m = _spmap(d3, map98_cols, map98_rows, map98_vals, N9, p['map'])
    d2 = _block(d3m, ed9, dis9, p['dec2'], N9, skip=e2)
    d2m = _spmap(d2, map109_cols, map109_rows, map109_vals, N10, p['map'])
    d1 = _block(d2m, ed10, dis10, p['dec1'], N10, skip=e1)

    emb10 = _rowwise(d1, W=p['out10']['W'], b=p['out10']['b'],
                     g=p['out10']['g'], beta=p['out10']['beta'], l2=True)
    emb9 = _rowwise(d2, W=p['out9']['W'], b=p['out9']['b'],
                    g=p['out9']['g'], beta=p['out9']['beta'], l2=True)
    emb8 = _rowwise(d3, W=p['out8']['W'], b=p['out8']['b'],
                    g=p['out8']['g'], beta=p['out8']['beta'], l2=True)
    recs = []
    for name in ('rec_poi', 'rec_mobility'):
        r = p[name]
        h = _rowwise(emb10, W=r['W1'], b=r['b1'], g=r['g1'], beta=r['beta1'],
                     gelu=True)
        recs.append(_rowwise(h, W=r['W2'], b=r['b2'], g=r['g2'],
                             beta=r['beta2'], l2=True))
    return (emb10, emb9, emb8, recs[0], recs[1])


# speedup vs baseline: 1.0010x; 1.0010x over previous
import functools
import math

import jax
import jax.numpy as jnp
from jax import lax
from jax.experimental import pallas as pl
from jax.experimental.pallas import tpu as pltpu
from jax.experimental.pallas import tpu_sc as plsc

_ROW_BLOCK = 512
_SQRT2 = math.sqrt(2.0)
_N8 = 2000


def _gelu_exact(y):
    return 0.5 * y * (1.0 + lax.erf(y / _SQRT2))


def _ln_rows(y, g, b, eps=1e-5):
    m = jnp.mean(y, axis=-1, keepdims=True)
    v = jnp.mean((y - m) ** 2, axis=-1, keepdims=True)
    return (y - m) / jnp.sqrt(v + eps) * g + b


def _l2_rows(y, eps=1e-8):
    n = jnp.sqrt(jnp.sum(y * y, axis=-1, keepdims=True))
    return y / jnp.maximum(n, eps)


def _rowwise_body(flags, *refs):
    it = iter(refs)
    x_ref = next(it)
    y = x_ref[...]
    if flags["nan"]:
        y = jnp.nan_to_num(y)
    if flags["skip"]:
        y = y + next(it)[...]
    if flags["rs_in"]:
        y = y * next(it)[...]
    if flags["matmul"]:
        w = next(it)[...]
        y = jnp.dot(y, w, preferred_element_type=jnp.float32)
    if flags["bias"]:
        y = y + next(it)[...]
    if flags["ln"]:
        g = next(it)[...]
        beta = next(it)[...]
        y = _ln_rows(y, g, beta)
    if flags["gelu"]:
        y = _gelu_exact(y)
    if flags["rs_out"]:
        y = y * next(it)[...]
    if flags["l2"]:
        y = _l2_rows(y)
    out_ref = next(it)
    out_ref[...] = y


def _rowwise(x, *, skip=None, rs_in=None, W=None, b=None, g=None, beta=None,
             rs_out=None, nan=False, gelu=False, l2=False):
    n, k = x.shape
    m = W.shape[1] if W is not None else k
    flags = dict(nan=nan, skip=skip is not None, rs_in=rs_in is not None,
                 matmul=W is not None, bias=b is not None,
                 ln=g is not None, gelu=gelu, rs_out=rs_out is not None,
                 l2=l2)
    args = [x]
    specs = [pl.BlockSpec((_ROW_BLOCK, k), lambda i: (i, 0))]
    if skip is not None:
        args.append(skip)
        specs.append(pl.BlockSpec((_ROW_BLOCK, k), lambda i: (i, 0)))
    if rs_in is not None:
        args.append(rs_in.reshape(n, 1))
        specs.append(pl.BlockSpec((_ROW_BLOCK, 1), lambda i: (i, 0)))
    if W is not None:
        args.append(W)
        specs.append(pl.BlockSpec((k, m), lambda i: (0, 0)))
    if b is not None:
        args.append(b.reshape(1, m))
        specs.append(pl.BlockSpec((1, m), lambda i: (0, 0)))
    if g is not None:
        args.append(g.reshape(1, m))
        specs.append(pl.BlockSpec((1, m), lambda i: (0, 0)))
        args.append(beta.reshape(1, m))
        specs.append(pl.BlockSpec((1, m), lambda i: (0, 0)))
    if rs_out is not None:
        args.append(rs_out.reshape(n, 1))
        specs.append(pl.BlockSpec((_ROW_BLOCK, 1), lambda i: (i, 0)))
    return pl.pallas_call(
        functools.partial(_rowwise_body, flags),
        grid=(pl.cdiv(n, _ROW_BLOCK),),
        in_specs=specs,
        out_specs=pl.BlockSpec((_ROW_BLOCK, m), lambda i: (i, 0)),
        out_shape=jax.ShapeDtypeStruct((n, m), jnp.float32),
    )(*args)


def _combine_body(w_ref, a_ref, b_ref, o_ref):
    y = a_ref[...] * w_ref[0] + b_ref[...] * w_ref[1]
    o_ref[...] = _l2_rows(y)


def _combine2_l2(a, b, w):
    n, k = a.shape
    return pl.pallas_call(
        _combine_body,
        grid=(pl.cdiv(n, _ROW_BLOCK),),
        in_specs=[pl.BlockSpec(memory_space=pltpu.SMEM),
                  pl.BlockSpec((_ROW_BLOCK, k), lambda i: (i, 0)),
                  pl.BlockSpec((_ROW_BLOCK, k), lambda i: (i, 0))],
        out_specs=pl.BlockSpec((_ROW_BLOCK, k), lambda i: (i, 0)),
        out_shape=jax.ShapeDtypeStruct((n, k), jnp.float32),
    )(w, a, b)


_NS = 16
_NC = 2
_JB = 128


def _acc_rows(n_out):
    return -(-(n_out + 1) // 128) * 128


def _pick_ke(n_out):
    budget = 2097000 - _acc_rows(n_out) * 32
    ke = budget // (_NS * 35) // _JB * _JB
    return max(_JB, min(2048, ke))


def _make_edge_kernel(n_in, epad, acc_rows, ke):
    stripe = acc_rows // _NS
    ept = epad // _NS
    nchunks = ept // ke
    nj = ke // _JB
    mesh = plsc.VectorSubcoreMesh(core_axis_name="c", subcore_axis_name="s")

    def body(xs, src2, dst2, ewf, zrow,
             out, idx_v, dst_v, ew_v, rows_v, acc, sem):
        c = lax.axis_index("c")
        s = lax.axis_index("s")
        off = c
        pltpu.sync_copy(zrow, acc.at[pl.ds(s * stripe, stripe)])
        plsc.subcore_barrier()

        def chunk(i, carry):
            b128 = s * (ept // _JB) + i * nj
            bflat = s * ept + i * ke
            stage = [pltpu.async_copy(src2.at[pl.ds(b128, nj)], idx_v, sem),
                     pltpu.async_copy(dst2.at[pl.ds(b128, nj)], dst_v, sem),
                     pltpu.async_copy(ewf.at[pl.ds(bflat, ke)], ew_v, sem)]
            for cp in stage:
                cp.wait()

            @plsc.parallel_loop(0, nj * (_JB // 16), unroll=4)
            def _(v):
                j = v // (_JB // 16)
                sl = pl.ds((v % (_JB // 16)) * 16, 16)
                idx_v[j, sl] = idx_v[j, sl] + off

            cps = [pltpu.async_copy(xs.at[idx_v.at[j]],
                                    rows_v.at[pl.ds(j * _JB, _JB)], sem)
                   for j in range(nj)]
            for cp in cps:
                cp.wait()

            @plsc.parallel_loop(0, ke, unroll=16)
            def _(e):
                sp = plsc.load_gather(ew_v, [jnp.full((16,), e, jnp.int32)])
                rows_v[e, 0:16] = rows_v[e, 0:16] * sp
                rows_v[e, 16:32] = rows_v[e, 16:32] * sp

            sps = [pltpu.async_copy(rows_v.at[pl.ds(j * _JB, _JB)],
                                    acc.at[dst_v.at[j]], sem, add=True)
                   for j in range(nj)]
            for sp in sps:
                sp.wait()
            return carry
        lax.fori_loop(0, nchunks, chunk, 0)
        plsc.subcore_barrier()
        pltpu.sync_copy(acc.at[pl.ds(s * stripe, stripe)],
                        out.at[c].at[pl.ds(s * stripe, stripe)])

    return pl.kernel(
        body, mesh=mesh,
        compiler_params=pltpu.CompilerParams(needs_layout_passes=False,
                                             use_tc_tiling_on_sc=False),
        out_type=jax.ShapeDtypeStruct((_NC, acc_rows, 32), jnp.float32),
        scratch_types=[
            pltpu.VMEM((nj, _JB), jnp.int32),
            pltpu.VMEM((nj, _JB), jnp.int32),
            pltpu.VMEM((ke,), jnp.float32),
            pltpu.VMEM((ke, 32), jnp.float32),
            pltpu.VMEM_SHARED((acc_rows, 32), jnp.float32),
            pltpu.SemaphoreType.DMA,
        ])


def _prep_edges(src, dst, w, n_out):
    e = src.shape[0]
    quant = _NS * _pick_ke(n_out)
    epad = -(-e // quant) * quant
    pad = epad - e
    src2 = jnp.pad(2 * src, (0, pad)).reshape(-1, _JB)
    dst2 = jnp.pad(dst, (0, pad), constant_values=n_out).reshape(-1, _JB)
    wf = jnp.pad(w, (0, pad))
    return src2, dst2, wf


def _edge_pass(x, src2, dst2, wf, n_out):
    n_in = x.shape[0]
    acc_rows = _acc_rows(n_out)
    zrow = jnp.zeros((acc_rows // _NS, 32), jnp.float32)
    xs = x.reshape(2 * n_in, 32)
    out = _make_edge_kernel(n_in, wf.shape[0], acc_rows, _pick_ke(n_out))(
        xs, src2, dst2, wf, zrow)
    return jnp.concatenate([out[0, :n_out], out[1, :n_out]], axis=1)


def _degree(edges, n):
    src2, dst2, wf = edges
    ones = jnp.ones((n, 64), jnp.float32)
    return _edge_pass(ones, src2, dst2, wf, n)[:, 0]


def _dis(deg):
    return jnp.where(deg > 0, lax.rsqrt(jnp.maximum(deg, 1e-30)), 0.0)


def _block(x, edges, dis, p, n, skip=None):
    src2, dst2, wf = edges
    inp = x if skip is None else x + skip
    idt = _rowwise(inp, W=p['rW'], b=p['rb'], g=p['rg'], beta=p['rbeta'],
                   gelu=True)
    out = inp
    for i in range(4):
        y = _rowwise(out, W=p['cW'][i], rs_out=dis)
        z = _edge_pass(y, src2, dst2, wf, n)
        out = _rowwise(z, rs_in=dis, b=p['cb'][i], g=p['ng'][i],
                       beta=p['nb'][i], gelu=True)
    return _rowwise(out, skip=idt, l2=True)


def _spmap(x, src, dst, vals, n_out, p):
    src2, dst2, wf = _prep_edges(src, dst, vals, n_out)
    mapped = _edge_pass(x, src2, dst2, wf, n_out)
    return _rowwise(mapped, W=p['W'], b=p['b'], gelu=True, l2=True)


def kernel(feat_poi, feat_mobility, edge_index_10, edge_weight_10,
           edge_index_9, edge_weight_9, edge_index_8, edge_weight_8,
           map109_rows, map109_cols, map109_vals, map98_rows, map98_cols,
           map98_vals, params):
    p = params
    f = p['fusion']
    N10 = feat_poi.shape[0]
    N9 = map98_rows.shape[0]
    N8 = _N8

    row10, col10 = edge_index_10[0], edge_index_10[1]
    row9, col9 = edge_index_9[0], edge_index_9[1]
    row8, col8 = edge_index_8[0], edge_index_8[1]

    ed10 = _prep_edges(row10, col10, edge_weight_10, N10)
    ed9 = _prep_edges(row9, col9, edge_weight_9, N9)
    ed8 = _prep_edges(row8, col8, edge_weight_8, N8)

    dis10 = _dis(_degree(ed10, N10))
    dis9 = _dis(_degree(ed9, N9))
    dis8 = _dis(_degree(ed8, N8))

    proj_p = _rowwise(feat_poi, W=f['poi_W'], b=f['poi_b'], g=f['poi_g'],
                      beta=f['poi_beta'], nan=True, gelu=True, l2=True)
    proj_m = _rowwise(feat_mobility, W=f['mob_W'], b=f['mob_b'], g=f['mob_g'],
                      beta=f['mob_beta'], nan=True, gelu=True, l2=True)
    w = jax.nn.softmax(f['mw'])
    x = _combine2_l2(proj_p, proj_m, w)

    e1 = _block(x, ed10, dis10, p['enc1'], N10)
    e1m = _spmap(e1, map109_rows, map109_cols, map109_vals, N9, p['map'])
    e2 = _block(e1m, ed9, dis9, p['enc2'], N9)
    e2m = _spmap(e2, map98_rows, map98_cols, map98_vals, N8, p['map'])
    e3 = _block(e2m, ed8, dis8, p['enc3'], N8)
    d3 = _block(e3, ed8, dis8, p['dec3'], N8, skip=e3)
    d3m = _spmap(d3, map98_cols, map98_rows, map98_vals, N9, p['map'])
    d2 = _block(d3m, ed9, dis9, p['dec2'], N9, skip=e2)
    d2m = _spmap(d2, map109_cols, map109_rows, map109_vals, N10, p['map'])
    d1 = _block(d2m, ed10, dis10, p['dec1'], N10, skip=e1)

    emb10 = _rowwise(d1, W=p['out10']['W'], b=p['out10']['b'],
                     g=p['out10']['g'], beta=p['out10']['beta'], l2=True)
    emb9 = _rowwise(d2, W=p['out9']['W'], b=p['out9']['b'],
                    g=p['out9']['g'], beta=p['out9']['beta'], l2=True)
    emb8 = _rowwise(d3, W=p['out8']['W'], b=p['out8']['b'],
                    g=p['out8']['g'], beta=p['out8']['beta'], l2=True)
    recs = []
    for name in ('rec_poi', 'rec_mobility'):
        r = p[name]
        h = _rowwise(emb10, W=r['W1'], b=r['b1'], g=r['g1'], beta=r['beta1'],
                     gelu=True)
        recs.append(_rowwise(h, W=r['W2'], b=r['b2'], g=r['g2'],
                             beta=r['beta2'], l2=True))
    return (emb10, emb9, emb8, recs[0], recs[1])
